# baseline (speedup 1.0000x reference)
import functools
import math

import jax
import jax.numpy as jnp
from jax.experimental import pallas as pl
from jax.experimental.pallas import tpu as pltpu

D_INP = 36
D_OB = 4
T = 215
B = 128
D_PE = 16
MAXV = 100.0
NHEAD = 4
D = D_INP * D_OB + D_PE
NHID = 128
NLAYERS = 2
DSTATIC = 9
NCLS = 2
DFINAL = D + D_INP
G = T * D_OB

BLK = 8
N = D_INP
BN = BLK * N


def _bdot(a, b, dims=None):
    a16 = a.astype(jnp.bfloat16)
    b16 = b.astype(jnp.bfloat16)
    if dims is None:
        return jnp.dot(a16, b16, preferred_element_type=jnp.float32)
    return jax.lax.dot_general(a16, b16, dims,
                               preferred_element_type=jnp.float32)


def _gat_kernel(x_ref, scale_ref, w1_ref, as1_ref, ad1_ref,
                w2_ref, as2_ref, ad2_ref, bm_ref, out_ref):
    bm = bm_ref[...]
    x = jnp.maximum(x_ref[...] * scale_ref[...], 0.0)

    def attn(h, a_s_row, a_d_row, ew):
        ss = jnp.sum(h * a_s_row, axis=1, keepdims=True)
        dd = jax.lax.dot_general(a_d_row, h, (((1,), (1,)), ((), ())),
                                 preferred_element_type=jnp.float32)
        s = ss + dd
        s = jnp.where(s >= 0, s, 0.2 * s)
        if ew is not None:
            s = s * ew
        sm = jnp.where(bm > 0, s, -1e30)
        mx = jnp.max(sm, axis=0, keepdims=True)
        e = jnp.exp(sm - mx) * bm
        den = jnp.sum(e, axis=0, keepdims=True)
        alpha = e / (den + 1e-16)
        out = _bdot(alpha, h, (((0,), (0,)), ((), ())))
        return out, alpha

    h1 = _bdot(x, w1_ref[...])
    o1, a1 = attn(h1, as1_ref[...], ad1_ref[...], None)
    h2 = _bdot(o1, w2_ref[...])
    o2, _ = attn(h2, as2_ref[...], ad2_ref[...], a1)
    out_ref[...] = o2


def _tfm_kernel(len_ref, r_ref, wq_ref, bq_ref, wk_ref, bk_ref, wv_ref,
                bv_ref, wo_ref, bo_ref, w1_ref, b1_ref, w2_ref, b2_ref,
                l1g_ref, l1b_ref, l2g_ref, l2b_ref, out_ref):
    b = pl.program_id(0)
    L = len_ref[b]
    r = r_ref[0]
    colmask = jax.lax.broadcasted_iota(jnp.int32, (1, T), 1) >= L
    dh = D // NHEAD
    inv_sqrt_dh = 1.0 / math.sqrt(dh)

    def layer_norm(x, g, bb):
        m = jnp.mean(x, axis=-1, keepdims=True)
        v = jnp.mean((x - m) ** 2, axis=-1, keepdims=True)
        return (x - m) / jnp.sqrt(v + 1e-5) * g + bb

    for l in range(NLAYERS):
        q = _bdot(r, wq_ref[l]) + bq_ref[l:l + 1]
        k = _bdot(r, wk_ref[l]) + bk_ref[l:l + 1]
        v = _bdot(r, wv_ref[l]) + bv_ref[l:l + 1]
        heads = []
        for h in range(NHEAD):
            qh = jax.lax.slice(q, (0, h * dh), (T, (h + 1) * dh))
            kh = jax.lax.slice(k, (0, h * dh), (T, (h + 1) * dh))
            vh = jax.lax.slice(v, (0, h * dh), (T, (h + 1) * dh))
            sc = _bdot(qh, kh, (((1,), (1,)), ((), ())))
            sc = sc * inv_sqrt_dh
            sc = jnp.where(colmask, -1e9, sc)
            mx = jnp.max(sc, axis=-1, keepdims=True)
            e = jnp.exp(sc - mx)
            at = e / jnp.sum(e, axis=-1, keepdims=True)
            heads.append(_bdot(at, vh))
        o = jnp.concatenate(heads, axis=1)
        o = _bdot(o, wo_ref[l]) + bo_ref[l:l + 1]
        r = layer_norm(r + o, l1g_ref[l:l + 1], l1b_ref[l:l + 1])
        ff = jnp.maximum(_bdot(r, w1_ref[l]) + b1_ref[l:l + 1], 0.0)
        ff = _bdot(ff, w2_ref[l]) + b2_ref[l:l + 1]
        r = layer_norm(r + ff, l2g_ref[l:l + 1], l2b_ref[l:l + 1])

    tmask = (jax.lax.broadcasted_iota(jnp.int32, (T, 1), 0) < L).astype(jnp.float32)
    pooled = jnp.sum(r * tmask, axis=0, keepdims=True)
    out_ref[0] = pooled / (L.astype(jnp.float32) + 1.0)


def _head_kernel(pooled_ref, static_ref, embw_ref, embb_ref,
                 w1_ref, b1_ref, w2_ref, b2_ref, out_ref):
    emb = jnp.dot(static_ref[...], embw_ref[...],
                  preferred_element_type=jnp.float32) + embb_ref[...]
    feat = jnp.concatenate([pooled_ref[...], emb], axis=1)
    hmid = jnp.maximum(
        jnp.dot(feat, w1_ref[...], preferred_element_type=jnp.float32)
        + b1_ref[...], 0.0)
    out_ref[...] = jnp.dot(hmid, w2_ref[...],
                           preferred_element_type=jnp.float32) + b2_ref[...]


def _full(spec_shape=None):
    return pl.BlockSpec(spec_shape, lambda *_: tuple(0 for _ in spec_shape)) \
        if spec_shape else None


@jax.jit
def kernel(src, static, times, lengths, R_u, emb_W, emb_b, g1_W, g1_asrc,
           g1_adst, g2_W, g2_asrc, g2_adst, t_Wq, t_bq, t_Wk, t_bk, t_Wv,
           t_bv, t_Wo, t_bo, t_W1, t_b1, t_W2, t_b2, ln1_g, ln1_b, ln2_g,
           ln2_b, mlp_W1, mlp_b1, mlp_W2, mlp_b2):
    f32 = jnp.float32

    x0 = src[:, :, :D_INP].transpose(1, 2, 0)
    xrep = jnp.broadcast_to(x0[..., None], (B, D_INP, T, D_OB))
    x2d = xrep.reshape(B * D_INP, G)
    scale = jnp.tile(R_u.reshape(D_INP, D_OB), (BLK, T))

    ids = jnp.arange(BN, dtype=jnp.int32) // N
    bm = (ids[:, None] == ids[None, :]).astype(f32)

    cparams = pltpu.CompilerParams(dimension_semantics=("arbitrary",))

    gat_out = pl.pallas_call(
        _gat_kernel,
        grid=(B // BLK,),
        in_specs=[
            pl.BlockSpec((BN, G), lambda i: (i, 0)),
            _full((BN, G)),
            _full((G, G)),
            _full((1, G)),
            _full((1, G)),
            _full((G, G)),
            _full((1, G)),
            _full((1, G)),
            _full((BN, BN)),
        ],
        out_specs=pl.BlockSpec((BN, G), lambda i: (i, 0)),
        out_shape=jax.ShapeDtypeStruct((B * D_INP, G), f32),
        compiler_params=cparams,
    )(x2d, scale, g1_W, g1_asrc.reshape(1, G), g1_adst.reshape(1, G),
      g2_W, g2_asrc.reshape(1, G), g2_adst.reshape(1, G), bm)

    gat_t = gat_out.reshape(B, D_INP, T, D_OB).transpose(0, 2, 1, 3)
    gat_t = gat_t.reshape(B, T, D_INP * D_OB)
    timescales = (float(T) ** jnp.linspace(0.0, 1.0, D_PE // 2)) * MAXV
    scaled = times.transpose(1, 0)[:, :, None] / timescales[None, None, :]
    pe = jnp.concatenate([jnp.sin(scaled), jnp.cos(scaled)], axis=-1)
    r0 = jnp.concatenate([gat_t, pe], axis=2)

    pooled = pl.pallas_call(
        _tfm_kernel,
        grid=(B,),
        in_specs=[
            pl.BlockSpec(memory_space=pltpu.SMEM),
            pl.BlockSpec((1, T, D), lambda i: (i, 0, 0)),
            _full((NLAYERS, D, D)), _full((NLAYERS, D)),
            _full((NLAYERS, D, D)), _full((NLAYERS, D)),
            _full((NLAYERS, D, D)), _full((NLAYERS, D)),
            _full((NLAYERS, D, D)), _full((NLAYERS, D)),
            _full((NLAYERS, D, NHID)), _full((NLAYERS, NHID)),
            _full((NLAYERS, NHID, D)), _full((NLAYERS, D)),
            _full((NLAYERS, D)), _full((NLAYERS, D)),
            _full((NLAYERS, D)), _full((NLAYERS, D)),
        ],
        out_specs=pl.BlockSpec((1, 1, D), lambda i: (i, 0, 0)),
        out_shape=jax.ShapeDtypeStruct((B, 1, D), f32),
        compiler_params=cparams,
    )(lengths, r0, t_Wq, t_bq, t_Wk, t_bk, t_Wv, t_bv, t_Wo, t_bo,
      t_W1, t_b1, t_W2, t_b2, ln1_g, ln1_b, ln2_g, ln2_b)
    pooled = pooled.reshape(B, D)

    logits = pl.pallas_call(
        _head_kernel,
        grid=(1,),
        in_specs=[
            _full((B, D)), _full((B, DSTATIC)),
            _full((DSTATIC, D_INP)), _full((1, D_INP)),
            _full((DFINAL, DFINAL)), _full((1, DFINAL)),
            _full((DFINAL, NCLS)), _full((1, NCLS)),
        ],
        out_specs=_full((B, NCLS)),
        out_shape=jax.ShapeDtypeStruct((B, NCLS), f32),
        compiler_params=cparams,
    )(pooled, static, emb_W, emb_b.reshape(1, D_INP),
      mlp_W1, mlp_b1.reshape(1, DFINAL), mlp_W2, mlp_b2.reshape(1, NCLS))

    return logits


# speedup vs baseline: 1.6910x; 1.6910x over previous
import math

import jax
import jax.numpy as jnp
from jax.experimental import pallas as pl
from jax.experimental.pallas import tpu as pltpu

D_INP = 36
D_OB = 4
T = 215
B = 128
D_PE = 16
MAXV = 100.0
NHEAD = 4
D = D_INP * D_OB + D_PE
NHID = 128
NLAYERS = 2
DSTATIC = 9
NCLS = 2
DFINAL = D + D_INP
G = T * D_OB

BLK = 8
N = D_INP
BN = BLK * N


def _bdot(a, b, dims=None):
    a16 = a.astype(jnp.bfloat16)
    b16 = b.astype(jnp.bfloat16)
    if dims is None:
        return jnp.dot(a16, b16, preferred_element_type=jnp.float32)
    return jax.lax.dot_general(a16, b16, dims,
                               preferred_element_type=jnp.float32)


def _gat_kernel(x_ref, scale_ref, w1_ref, as1_ref, ad1_ref,
                w2_ref, as2_ref, ad2_ref, bm_ref, out_ref):
    bm = bm_ref[...]
    xs = x_ref[...]
    xt = jnp.concatenate([xs, xs, xs, xs], axis=1)
    x = jnp.maximum(xt * scale_ref[...], 0.0)

    def attn(h, a_s_row, a_d_row, ew):
        ss = jnp.sum(h * a_s_row, axis=1, keepdims=True)
        dd = jax.lax.dot_general(a_d_row, h, (((1,), (1,)), ((), ())),
                                 preferred_element_type=jnp.float32)
        s = ss + dd
        s = jnp.where(s >= 0, s, 0.2 * s)
        if ew is not None:
            s = s * ew
        sm = jnp.where(bm > 0, s, -1e30)
        mx = jnp.max(sm, axis=0, keepdims=True)
        e = jnp.exp(sm - mx) * bm
        den = jnp.sum(e, axis=0, keepdims=True)
        alpha = e / (den + 1e-16)
        out = _bdot(alpha, h, (((0,), (0,)), ((), ())))
        return out, alpha

    h1 = _bdot(x, w1_ref[...])
    o1, a1 = attn(h1, as1_ref[...], ad1_ref[...], None)
    h2 = _bdot(o1, w2_ref[...])
    o2, _ = attn(h2, as2_ref[...], ad2_ref[...], a1)
    o16 = o2.astype(jnp.bfloat16)
    for b in range(BLK):
        out_ref[b] = o16[b * N:(b + 1) * N, :]


def _tfm_kernel(len_ref, gat_ref, pe_ref, eye_ref, wq_ref, bq_ref, wk_ref,
                bk_ref, wv_ref, bv_ref, wo_ref, bo_ref, w1_ref, b1_ref,
                w2_ref, b2_ref, l1g_ref, l1b_ref, l2g_ref, l2b_ref, out_ref):
    b = pl.program_id(0)
    L = len_ref[b]
    go = gat_ref[0]
    goT = jax.lax.dot_general(
        go.astype(jnp.float32), eye_ref[...], (((0,), (0,)), ((), ())),
        preferred_element_type=jnp.float32)
    slabs = [goT[j * T:(j + 1) * T, :] for j in range(D_OB)]
    r = jnp.concatenate(slabs + [pe_ref[0]], axis=1)

    colmask = jax.lax.broadcasted_iota(jnp.int32, (1, T), 1) >= L
    dh = D // NHEAD
    inv_sqrt_dh = 1.0 / math.sqrt(dh)

    def layer_norm(x, g, bb):
        m = jnp.mean(x, axis=-1, keepdims=True)
        v = jnp.mean((x - m) ** 2, axis=-1, keepdims=True)
        return (x - m) / jnp.sqrt(v + 1e-5) * g + bb

    for l in range(NLAYERS):
        q = _bdot(r, wq_ref[l]) + bq_ref[l:l + 1]
        k = _bdot(r, wk_ref[l]) + bk_ref[l:l + 1]
        v = _bdot(r, wv_ref[l]) + bv_ref[l:l + 1]
        heads = []
        for h in range(NHEAD):
            qh = jax.lax.slice(q, (0, h * dh), (T, (h + 1) * dh))
            kh = jax.lax.slice(k, (0, h * dh), (T, (h + 1) * dh))
            vh = jax.lax.slice(v, (0, h * dh), (T, (h + 1) * dh))
            sc = _bdot(qh, kh, (((1,), (1,)), ((), ())))
            sc = sc * inv_sqrt_dh
            sc = jnp.where(colmask, -1e9, sc)
            mx = jnp.max(sc, axis=-1, keepdims=True)
            e = jnp.exp(sc - mx)
            at = e / jnp.sum(e, axis=-1, keepdims=True)
            heads.append(_bdot(at, vh))
        o = jnp.concatenate(heads, axis=1)
        o = _bdot(o, wo_ref[l]) + bo_ref[l:l + 1]
        r = layer_norm(r + o, l1g_ref[l:l + 1], l1b_ref[l:l + 1])
        ff = jnp.maximum(_bdot(r, w1_ref[l]) + b1_ref[l:l + 1], 0.0)
        ff = _bdot(ff, w2_ref[l]) + b2_ref[l:l + 1]
        r = layer_norm(r + ff, l2g_ref[l:l + 1], l2b_ref[l:l + 1])

    tmask = (jax.lax.broadcasted_iota(jnp.int32, (T, 1), 0) < L).astype(jnp.float32)
    pooled = jnp.sum(r * tmask, axis=0, keepdims=True)
    out_ref[0] = pooled / (L.astype(jnp.float32) + 1.0)


def _head_kernel(pooled_ref, static_ref, embw_ref, embb_ref,
                 w1_ref, b1_ref, w2_ref, b2_ref, out_ref):
    emb = jnp.dot(static_ref[...], embw_ref[...],
                  preferred_element_type=jnp.float32) + embb_ref[...]
    feat = jnp.concatenate([pooled_ref[...], emb], axis=1)
    hmid = jnp.maximum(
        jnp.dot(feat, w1_ref[...], preferred_element_type=jnp.float32)
        + b1_ref[...], 0.0)
    out_ref[...] = jnp.dot(hmid, w2_ref[...],
                           preferred_element_type=jnp.float32) + b2_ref[...]


def _full(spec_shape):
    return pl.BlockSpec(spec_shape, lambda *_: tuple(0 for _ in spec_shape))


@jax.jit
def kernel(src, static, times, lengths, R_u, emb_W, emb_b, g1_W, g1_asrc,
           g1_adst, g2_W, g2_asrc, g2_adst, t_Wq, t_bq, t_Wk, t_bk, t_Wv,
           t_bv, t_Wo, t_bo, t_W1, t_b1, t_W2, t_b2, ln1_g, ln1_b, ln2_g,
           ln2_b, mlp_W1, mlp_b1, mlp_W2, mlp_b2):
    f32 = jnp.float32
    bf16 = jnp.bfloat16

    gg = jnp.arange(G)
    perm_g = (gg % T) * D_OB + gg // T
    f2 = jnp.arange(D_INP * D_OB)
    permF = jnp.concatenate([(f2 % D_INP) * D_OB + f2 // D_INP,
                             jnp.arange(D_INP * D_OB, D)])
    permF196 = jnp.concatenate([permF, jnp.arange(D, DFINAL)])

    g1_W2 = g1_W[perm_g][:, perm_g].astype(bf16)
    g2_W2 = g2_W[perm_g][:, perm_g].astype(bf16)
    g1_as = g1_asrc[perm_g].reshape(1, G)
    g1_ad = g1_adst[perm_g].reshape(1, G)
    g2_as = g2_asrc[perm_g].reshape(1, G)
    g2_ad = g2_adst[perm_g].reshape(1, G)

    wq = t_Wq[:, permF, :].astype(bf16)
    wk = t_Wk[:, permF, :].astype(bf16)
    wv = t_Wv[:, permF, :].astype(bf16)
    w1 = t_W1[:, permF, :].astype(bf16)
    wo = t_Wo[:, :, permF].astype(bf16)
    w2 = t_W2[:, :, permF].astype(bf16)
    bo = t_bo[:, permF]
    b2 = t_b2[:, permF]
    l1g = ln1_g[:, permF]
    l1b = ln1_b[:, permF]
    l2g = ln2_g[:, permF]
    l2b = ln2_b[:, permF]
    mw1 = mlp_W1[permF196, :]

    x0_2d = src[:, :, :D_INP].transpose(1, 2, 0).reshape(B * D_INP, T)
    scale = jnp.tile(jnp.repeat(R_u.reshape(D_INP, D_OB), T, axis=1), (BLK, 1))

    ids = jnp.arange(BN, dtype=jnp.int32) // N
    bm = (ids[:, None] == ids[None, :]).astype(f32)

    cparams = pltpu.CompilerParams(dimension_semantics=("arbitrary",))

    gat_out = pl.pallas_call(
        _gat_kernel,
        grid=(B // BLK,),
        in_specs=[
            pl.BlockSpec((BN, T), lambda i: (i, 0)),
            _full((BN, G)),
            _full((G, G)), _full((1, G)), _full((1, G)),
            _full((G, G)), _full((1, G)), _full((1, G)),
            _full((BN, BN)),
        ],
        out_specs=pl.BlockSpec((BLK, N, G), lambda i: (i, 0, 0)),
        out_shape=jax.ShapeDtypeStruct((B, N, G), bf16),
        compiler_params=cparams,
    )(x0_2d, scale, g1_W2, g1_as, g1_ad, g2_W2, g2_as, g2_ad, bm)

    timescales = (float(T) ** jnp.linspace(0.0, 1.0, D_PE // 2)) * MAXV
    scaled = times.transpose(1, 0)[:, :, None] / timescales[None, None, :]
    pe = jnp.concatenate([jnp.sin(scaled), jnp.cos(scaled)], axis=-1)

    pooled = pl.pallas_call(
        _tfm_kernel,
        grid=(B,),
        in_specs=[
            pl.BlockSpec(memory_space=pltpu.SMEM),
            pl.BlockSpec((1, N, G), lambda i: (i, 0, 0)),
            pl.BlockSpec((1, T, D_PE), lambda i: (i, 0, 0)),
            _full((N, N)),
            _full((NLAYERS, D, D)), _full((NLAYERS, D)),
            _full((NLAYERS, D, D)), _full((NLAYERS, D)),
            _full((NLAYERS, D, D)), _full((NLAYERS, D)),
            _full((NLAYERS, D, D)), _full((NLAYERS, D)),
            _full((NLAYERS, D, NHID)), _full((NLAYERS, NHID)),
            _full((NLAYERS, NHID, D)), _full((NLAYERS, D)),
            _full((NLAYERS, D)), _full((NLAYERS, D)),
            _full((NLAYERS, D)), _full((NLAYERS, D)),
        ],
        out_specs=pl.BlockSpec((1, 1, D), lambda i: (i, 0, 0)),
        out_shape=jax.ShapeDtypeStruct((B, 1, D), f32),
        compiler_params=cparams,
    )(lengths, gat_out, pe, jnp.eye(N, dtype=f32),
      wq, t_bq, wk, t_bk, wv, t_bv, wo, bo,
      w1, t_b1, w2, b2, l1g, l1b, l2g, l2b)
    pooled = pooled.reshape(B, D)

    logits = pl.pallas_call(
        _head_kernel,
        grid=(1,),
        in_specs=[
            _full((B, D)), _full((B, DSTATIC)),
            _full((DSTATIC, D_INP)), _full((1, D_INP)),
            _full((DFINAL, DFINAL)), _full((1, DFINAL)),
            _full((DFINAL, NCLS)), _full((1, NCLS)),
        ],
        out_specs=_full((B, NCLS)),
        out_shape=jax.ShapeDtypeStruct((B, NCLS), f32),
        compiler_params=cparams,
    )(pooled, static, emb_W, emb_b.reshape(1, D_INP),
      mw1, mlp_b1.reshape(1, DFINAL), mlp_W2, mlp_b2.reshape(1, NCLS))

    return logits
